# Initial kernel scaffold; baseline (speedup 1.0000x reference)
import jax
import jax.numpy as jnp
from jax.experimental import pallas as pl


def kernel(input_ids, segment_ids, token_table, position_table, segment_table, ln_gamma, ln_beta):
    raise NotImplementedError("write your pallas kernel here")


# speedup vs baseline: 1.9213x; 1.9213x over previous
import functools

import jax
import jax.numpy as jnp
from jax import lax
from jax.experimental import pallas as pl
from jax.experimental.pallas import tpu as pltpu
from jax.experimental.pallas import tpu_sc as plsc

D = 128
NUM_CORES = 2
NUM_SUBCORES = 16
NW = NUM_CORES * NUM_SUBCORES
IDX_CHUNK = 128


def _sc_gather(table, ids_flat, n_tokens):
    b_per_w = n_tokens // NW
    k = b_per_w // IDX_CHUNK
    ids_3d = ids_flat.reshape(NW, k, IDX_CHUNK)
    mesh = plsc.VectorSubcoreMesh(core_axis_name="c", subcore_axis_name="s")

    @functools.partial(
        pl.kernel,
        mesh=mesh,
        out_type=jax.ShapeDtypeStruct((n_tokens, D), jnp.float32),
        scratch_types=[
            pltpu.VMEM((k, IDX_CHUNK), jnp.int32),
            pltpu.VMEM((b_per_w, D), jnp.float32),
            pltpu.SemaphoreType.DMA,
        ],
    )
    def gather_kernel(table_hbm, idx_hbm, out_hbm, idx_v, rows_v, sem):
        wid = lax.axis_index("s") * NUM_CORES + lax.axis_index("c")
        pltpu.sync_copy(idx_hbm.at[wid], idx_v)
        copies = [
            pltpu.async_copy(
                table_hbm.at[idx_v.at[j]],
                rows_v.at[pl.ds(j * IDX_CHUNK, IDX_CHUNK)],
                sem,
            )
            for j in range(k)
        ]
        for c in copies:
            c.wait()
        pltpu.sync_copy(rows_v, out_hbm.at[pl.ds(wid * b_per_w, b_per_w)])

    return gather_kernel(table, ids_3d)


def _ln_body(gat_ref, pos_ref, segf_ref, segtab_ref, gam_ref, beta_ref, out_ref):
    x = gat_ref[0] + pos_ref[...]
    segf = segf_ref[0, 0, :]
    seg0 = segtab_ref[0, :]
    dseg = segtab_ref[1, :] - seg0
    x = x + seg0[None, :] + segf[:, None] * dseg[None, :]
    mean = jnp.mean(x, axis=-1, keepdims=True)
    var = jnp.mean((x - mean) ** 2, axis=-1, keepdims=True)
    xhat = (x - mean) * lax.rsqrt(var + 1e-5)
    out_ref[0] = xhat * gam_ref[...] + beta_ref[...]


def _tc_layernorm(gathered, position_table, seg_f, segment_table, gamma, beta):
    b, s, _ = gathered.shape
    return pl.pallas_call(
        _ln_body,
        grid=(b,),
        in_specs=[
            pl.BlockSpec((1, s, D), lambda i: (i, 0, 0)),
            pl.BlockSpec((s, D), lambda i: (0, 0)),
            pl.BlockSpec((1, 1, s), lambda i: (i, 0, 0)),
            pl.BlockSpec((2, D), lambda i: (0, 0)),
            pl.BlockSpec((1, D), lambda i: (0, 0)),
            pl.BlockSpec((1, D), lambda i: (0, 0)),
        ],
        out_specs=pl.BlockSpec((1, s, D), lambda i: (i, 0, 0)),
        out_shape=jax.ShapeDtypeStruct((b, s, D), jnp.float32),
    )(gathered, position_table, seg_f, segment_table, gamma, beta)


def kernel(input_ids, segment_ids, token_table, position_table, segment_table,
           ln_gamma, ln_beta):
    b, s = input_ids.shape
    n = b * s
    ids_flat = input_ids.reshape(n).astype(jnp.int32)
    gathered = _sc_gather(token_table, ids_flat, n)
    seg_f = segment_ids.astype(jnp.float32).reshape(b, 1, s)
    out = _tc_layernorm(
        gathered.reshape(b, s, D),
        position_table,
        seg_f,
        segment_table,
        ln_gamma.reshape(1, D),
        ln_beta.reshape(1, D),
    )
    return out
